# baseline (speedup 1.0000x reference)
import functools

import jax
import jax.numpy as jnp
from jax import lax
from jax.experimental import pallas as pl
from jax.experimental.pallas import tpu as pltpu
from jax.experimental.pallas import tpu_sc as plsc

_NUM_WORKERS = 32
_RING = 3


def kernel(inputs, token_table, pos_table):
    B, S = inputs.shape
    V, D = token_table.shape
    seq_per_w = B // _NUM_WORKERS
    sizes = (96, S - 96)
    offs = (0, 96)
    n_loop = (seq_per_w - 2) // _RING

    idx1 = inputs.astype(jnp.int32).reshape(-1)

    mesh = plsc.VectorSubcoreMesh(core_axis_name="c", subcore_axis_name="s")

    @functools.partial(
        pl.kernel,
        mesh=mesh,
        out_type=jax.ShapeDtypeStruct((B, S, D), jnp.float32),
        scratch_types=(
            [pltpu.VMEM((seq_per_w * S,), jnp.int32),
             pltpu.VMEM((S, D), jnp.float32)]
            + [pltpu.VMEM((S, D), jnp.float32)] * _RING
            + [pltpu.SemaphoreType.DMA] * (2 * _RING + 1)
        ),
    )
    def emb_kernel(idx_hbm, tok_hbm, pos_hbm, out_hbm, idx_v, pos_v, *rest):
        bufs = rest[:_RING]
        gsems = rest[_RING:2 * _RING]
        ssems = rest[2 * _RING:3 * _RING]
        psem = rest[3 * _RING]
        wid = lax.axis_index("s") * 2 + lax.axis_index("c")
        seq0 = wid * seq_per_w
        pltpu.sync_copy(idx_hbm.at[pl.ds(seq0 * S, seq_per_w * S)], idx_v)
        pos_cp = pltpu.async_copy(pos_hbm, pos_v, psem)

        def issue_gather(j, r):
            pltpu.async_copy(
                tok_hbm.at[idx_v.at[pl.ds(j * S, S)]], bufs[r], gsems[r])

        def wait_gather(r):
            pltpu.make_async_copy(
                tok_hbm.at[idx_v.at[pl.ds(0, S)]], bufs[r], gsems[r]).wait()

        def issue_store(j, r):
            pltpu.async_copy(bufs[r], out_hbm.at[seq0 + j], ssems[r])

        def drain_store(r):
            pltpu.make_async_copy(bufs[r], out_hbm.at[0], ssems[r]).wait()

        def add_pos(r):
            buf = bufs[r]

            def body(i, c):
                for k in range(8):
                    rr = i * 8 + k
                    for c8 in range(D // 16):
                        sl = pl.ds(c8 * 16, 16)
                        plsc.addupdate(buf.at[rr, sl], pos_v[rr, sl])
                return c
            lax.fori_loop(0, S // 8, body, 0)

        issue_gather(0, 0)
        pos_cp.wait()

        def step(t, carry):
            for k in range(_RING):
                j = t * _RING + k
                rn = (k + 1) % _RING
                if k < 2:
                    pl.when(t >= 1)(lambda r2=rn: drain_store(r2))
                else:
                    drain_store(rn)
                issue_gather(j + 1, rn)
                wait_gather(k)
                add_pos(k)
                issue_store(j, k)
            return carry

        lax.fori_loop(0, n_loop, step, 0)

        for j in range(n_loop * _RING, seq_per_w):
            r = j % _RING
            if j + 1 < seq_per_w:
                rn = (j + 1) % _RING
                drain_store(rn)
                issue_gather(j + 1, rn)
            wait_gather(r)
            add_pos(r)
            issue_store(j, r)
        for j in range(seq_per_w - _RING, seq_per_w):
            drain_store(j % _RING)

    return emb_kernel(idx1, token_table, pos_table)


# speedup vs baseline: 1.0114x; 1.0114x over previous
import functools

import jax
import jax.numpy as jnp
from jax import lax
from jax.experimental import pallas as pl
from jax.experimental.pallas import tpu as pltpu
from jax.experimental.pallas import tpu_sc as plsc

_NUM_WORKERS = 32
_RING = 4
_LEAD = 2


def kernel(inputs, token_table, pos_table):
    B, S = inputs.shape
    V, D = token_table.shape
    seq_per_w = B // _NUM_WORKERS
    n_units = seq_per_w * 2
    sizes = (96, S - 96)
    offs = (0, 96)
    bufsz = max(sizes)
    n_loop = (n_units - _RING // 2) // _RING

    idx1 = inputs.astype(jnp.int32).reshape(-1)

    mesh = plsc.VectorSubcoreMesh(core_axis_name="c", subcore_axis_name="s")

    @functools.partial(
        pl.kernel,
        mesh=mesh,
        out_type=jax.ShapeDtypeStruct((B, S, D), jnp.float32),
        scratch_types=(
            [pltpu.VMEM((seq_per_w * S,), jnp.int32),
             pltpu.VMEM((S, D), jnp.float32)]
            + [pltpu.VMEM((bufsz, D), jnp.float32)] * _RING
            + [pltpu.SemaphoreType.DMA] * (2 * _RING)
        ),
    )
    def emb_kernel(idx_hbm, tok_hbm, pos_hbm, out_hbm, idx_v, pos_v, *rest):
        bufs = rest[:_RING]
        gsems = rest[_RING:2 * _RING]
        ssems = rest[2 * _RING:]
        wid = lax.axis_index("s") * 2 + lax.axis_index("c")
        seq0 = wid * seq_per_w
        pltpu.sync_copy(idx_hbm.at[pl.ds(seq0 * S, seq_per_w * S)], idx_v)
        pltpu.sync_copy(pos_hbm, pos_v)

        def issue_gather(u, p, r):
            pltpu.async_copy(
                tok_hbm.at[idx_v.at[pl.ds((u // 2) * S + offs[p], sizes[p])]],
                bufs[r].at[pl.ds(0, sizes[p])], gsems[r])

        def wait_gather(p, r):
            pltpu.make_async_copy(
                tok_hbm.at[idx_v.at[pl.ds(offs[p], sizes[p])]],
                bufs[r].at[pl.ds(0, sizes[p])], gsems[r]).wait()

        def issue_store(u, p, r):
            pltpu.async_copy(
                bufs[r].at[pl.ds(0, sizes[p])],
                out_hbm.at[seq0 + u // 2, pl.ds(offs[p], sizes[p])],
                ssems[r])

        def drain_store(p, r):
            pltpu.make_async_copy(
                bufs[r].at[pl.ds(0, sizes[p])],
                out_hbm.at[0, pl.ds(offs[p], sizes[p])], ssems[r]).wait()

        def add_pos(p, r):
            buf = bufs[r]

            def body(i, c):
                for k in range(4):
                    rr = i * 4 + k
                    for c8 in range(D // 16):
                        sl = pl.ds(c8 * 16, 16)
                        plsc.addupdate(buf.at[rr, sl],
                                       pos_v[offs[p] + rr, sl])
                return c
            lax.fori_loop(0, sizes[p] // 4, body, 0)

        for u0 in range(_LEAD):
            issue_gather(u0, u0 % 2, u0)

        def step(t, carry):
            for k in range(_RING):
                u = t * _RING + k
                p = k % 2
                rg = (k + _LEAD) % _RING
                pg = (k + _LEAD) % 2
                if k < _LEAD:
                    pl.when(t >= 1)(lambda p2=pg, r2=rg: drain_store(p2, r2))
                else:
                    drain_store(pg, rg)
                issue_gather(u + _LEAD, pg, rg)
                wait_gather(p, k)
                add_pos(p, k)
                issue_store(u, p, k)
            return carry

        lax.fori_loop(0, n_loop, step, 0)

        for u in range(n_loop * _RING, n_units):
            r = u % _RING
            p = u % 2
            if u + _LEAD < n_units:
                rg = (u + _LEAD) % _RING
                pg = (u + _LEAD) % 2
                drain_store(pg, rg)
                issue_gather(u + _LEAD, pg, rg)
            wait_gather(p, r)
            add_pos(p, r)
            issue_store(u, p, r)
        for u in range(n_units - _RING, n_units):
            drain_store(u % 2, u % _RING)

    return emb_kernel(idx1, token_table, pos_table)
